# baseline (speedup 1.0000x reference)
import jax
import jax.numpy as jnp
from jax import lax
from jax.experimental import pallas as pl
from jax.experimental.pallas import tpu as pltpu
from jax.experimental.pallas import tpu_sc as plsc

B = 16384
L = 50
D = 32
CUST_V = 100001
NC = 2
NS = 16
NW = NC * NS
PER_W = B // NW
C = 32
NCHUNK = PER_W // C
INV_L = 1.0 / L

NBLK = CUST_V // 128
PK_ROWS = NBLK * 32 + 16
BLK_PER_W = (NBLK + NW - 1) // NW


def _conv_body(ctT_hbm, tailpk_hbm, ctpk_hbm, cbuf, obuf):
    wid = lax.axis_index("s") * NC + lax.axis_index("c")
    iota = lax.iota(jnp.int32, 16)

    def blk_body(i, _):
        blk = i * NW + wid

        @pl.when(blk < NBLK)
        def _():
            pltpu.sync_copy(ctT_hbm.at[:, pl.ds(blk * 128, 128)], cbuf)

            def col_body(j, _):
                jf = jnp.full((16,), j, jnp.int32)
                a = plsc.load_gather(cbuf, [iota, jf])
                b = plsc.load_gather(cbuf, [iota + 16, jf])
                r, q = j // 4, (j % 4) * 32
                obuf[r, pl.ds(q, 16)] = a
                obuf[r, pl.ds(q + 16, 16)] = b
                return 0

            lax.fori_loop(0, 128, col_body, 0)
            pltpu.sync_copy(obuf, ctpk_hbm.at[pl.ds(blk * 32, 32), :])

        return 0

    lax.fori_loop(0, BLK_PER_W, blk_body, 0)

    @pl.when(wid == 0)
    def _():
        pltpu.sync_copy(tailpk_hbm, ctpk_hbm.at[pl.ds(NBLK * 32, 16), :])


def _unpack2(x32):
    return plsc.unpack(x32, format=plsc.PackFormat.INTERLEAVED)


def _body(cname_hbm, tsubj_hbm, ctpk_hbm, ttab_hbm, out_hbm,
          cidx_v, cidx4_v, crow_v, tidx0, tidx1, rows0, rows1, outc_v,
          csem, sem0, sem1):
    wid = lax.axis_index("s") * NC + lax.axis_index("c")
    base = wid * PER_W

    pltpu.sync_copy(cname_hbm.at[pl.ds(base, PER_W)], cidx_v)

    def shift_body(i, _):
        cidx4_v[pl.ds(i * 16, 16)] = cidx_v[pl.ds(i * 16, 16)] // 4
        return 0

    lax.fori_loop(0, PER_W // 16, shift_body, 0)
    ccopy = pltpu.async_copy(ctpk_hbm.at[cidx4_v], crow_v, csem)

    def issue(k, tidx, rows, sem):
        pltpu.sync_copy(tsubj_hbm.at[pl.ds((base + k * C) * L, C * L)], tidx)
        return pltpu.async_copy(ttab_hbm.at[tidx], rows, sem)

    def reduce_chunk(k, rows):
        iota1 = lax.iota(jnp.int32, 16)
        iota2 = 2 * iota1

        def elem_body(e, _):
            r0 = e * L
            a0, a1 = _unpack2(rows[r0, pl.ds(0, D)])
            for l in range(1, L):
                b0, b1 = _unpack2(rows[r0 + l, pl.ds(0, D)])
                a0 = a0 + b0
                a1 = a1 + b1
            ce = k * C + e
            cef = jnp.full((16,), ce, jnp.int32)
            cidv = plsc.load_gather(cidx_v, [cef])
            qv = (cidv % 4) * 32 + iota1
            o = e * (2 * D)
            c0 = plsc.load_gather(crow_v, [cef, qv])
            c1 = plsc.load_gather(crow_v, [cef, qv + 16])
            outc_v[pl.ds(o, 16)] = c0
            outc_v[pl.ds(o + 16, 16)] = c1
            plsc.store_scatter(outc_v, [o + D + iota2], a0 * INV_L)
            plsc.store_scatter(outc_v, [o + D + 1 + iota2], a1 * INV_L)
            return 0

        lax.fori_loop(0, C, elem_body, 0)
        pltpu.sync_copy(outc_v, out_hbm.at[pl.ds((base + k * C) * 2 * D,
                                                 C * 2 * D)])

    issue(0, tidx0, rows0, sem0)
    ccopy.wait()

    def pair_body(p, _):
        ka = 2 * p
        issue(ka + 1, tidx1, rows1, sem1)
        pltpu.make_async_copy(ttab_hbm.at[tidx0], rows0, sem0).wait()
        reduce_chunk(ka, rows0)

        @pl.when(p < NCHUNK // 2 - 1)
        def _():
            issue(ka + 2, tidx0, rows0, sem0)

        pltpu.make_async_copy(ttab_hbm.at[tidx1], rows1, sem1).wait()
        reduce_chunk(ka + 1, rows1)
        return 0

    lax.fori_loop(0, NCHUNK // 2, pair_body, 0)


@jax.jit
def kernel(customer_name, ticket_subject, customer_table, ticket_table):
    mesh = plsc.VectorSubcoreMesh(core_axis_name="c", subcore_axis_name="s")

    tail_flat = jnp.reshape(customer_table[NBLK * 128:], (-1,))
    tailpk = jnp.reshape(
        jnp.concatenate([tail_flat,
                         jnp.zeros((16 * 128 - 33 * 32,), jnp.float32)]),
        (16, 128))

    conv = pl.kernel(
        _conv_body,
        out_type=jax.ShapeDtypeStruct((PK_ROWS, 128), jnp.float32),
        mesh=mesh,
        scratch_types=[
            pltpu.VMEM((D, 128), jnp.float32),
            pltpu.VMEM((32, 128), jnp.float32),
        ],
        compiler_params=pltpu.CompilerParams(use_tc_tiling_on_sc=True,
                                             needs_layout_passes=False),
    )
    ctpk = conv(customer_table.T, tailpk)

    tt_bf = ticket_table.astype(jnp.bfloat16)
    tsubj_flat = jnp.reshape(ticket_subject, (B * L,))
    k = pl.kernel(
        _body,
        out_type=jax.ShapeDtypeStruct((B * 2 * D,), jnp.float32),
        mesh=mesh,
        scratch_types=[
            pltpu.VMEM((PER_W,), jnp.int32),
            pltpu.VMEM((PER_W,), jnp.int32),
            pltpu.VMEM((PER_W, 128), jnp.float32),
            pltpu.VMEM((C * L,), jnp.int32),
            pltpu.VMEM((C * L,), jnp.int32),
            pltpu.VMEM((C * L, D), jnp.bfloat16),
            pltpu.VMEM((C * L, D), jnp.bfloat16),
            pltpu.VMEM((C * 2 * D,), jnp.float32),
            pltpu.SemaphoreType.DMA,
            pltpu.SemaphoreType.DMA,
            pltpu.SemaphoreType.DMA,
        ],
        compiler_params=pltpu.CompilerParams(use_tc_tiling_on_sc=False,
                                             needs_layout_passes=False),
    )
    out_flat = k(customer_name, tsubj_flat, ctpk, tt_bf)
    return jnp.reshape(out_flat, (B, 2 * D))


# speedup vs baseline: 1.1270x; 1.1270x over previous
import jax
import jax.numpy as jnp
from jax import lax
from jax.experimental import pallas as pl
from jax.experimental.pallas import tpu as pltpu
from jax.experimental.pallas import tpu_sc as plsc

B = 16384
L = 50
D = 32
CUST_V = 100001
NC = 2
NS = 16
NW = NC * NS
PER_W = B // NW
C = 32
NCHUNK = PER_W // C
INV_L = 1.0 / L

NBLK = CUST_V // 128
PK_ROWS = NBLK * 32 + 16
BLK_PER_W = (NBLK + NW - 1) // NW


def _conv_body(ctT_hbm, tailpk_hbm, ctpk_hbm, cbuf0, cbuf1, obuf, dsem0,
               dsem1):
    wid = lax.axis_index("s") * NC + lax.axis_index("c")
    iota = lax.iota(jnp.int32, 16)

    def start(i, cbuf, dsem):
        blk = i * NW + wid

        @pl.when(blk < NBLK)
        def _():
            pltpu.async_copy(ctT_hbm.at[:, pl.ds(blk * 128, 128)], cbuf, dsem)

    def process(i, cbuf, dsem):
        blk = i * NW + wid

        @pl.when(blk < NBLK)
        def _():
            pltpu.make_async_copy(
                ctT_hbm.at[:, pl.ds(blk * 128, 128)], cbuf, dsem).wait()

            def grp_body(g, _):
                for u in range(4):
                    jf = jnp.full((16,), g * 4 + u, jnp.int32)
                    a = plsc.load_gather(cbuf, [iota, jf])
                    b = plsc.load_gather(cbuf, [iota + 16, jf])
                    obuf[g, pl.ds(u * 32, 16)] = a
                    obuf[g, pl.ds(u * 32 + 16, 16)] = b
                return 0

            lax.fori_loop(0, 32, grp_body, 0)
            pltpu.sync_copy(obuf, ctpk_hbm.at[pl.ds(blk * 32, 32), :])

    start(0, cbuf0, dsem0)

    def pair_body(p, _):
        ia = 2 * p
        start(ia + 1, cbuf1, dsem1)
        process(ia, cbuf0, dsem0)

        @pl.when(p < BLK_PER_W // 2)
        def _():
            start(ia + 2, cbuf0, dsem0)

        process(ia + 1, cbuf1, dsem1)
        return 0

    lax.fori_loop(0, BLK_PER_W // 2, pair_body, 0)
    process(BLK_PER_W - 1, cbuf0, dsem0)

    @pl.when(wid == 0)
    def _():
        pltpu.sync_copy(tailpk_hbm, ctpk_hbm.at[pl.ds(NBLK * 32, 16), :])


def _unpack2(x32):
    return plsc.unpack(x32, format=plsc.PackFormat.INTERLEAVED)


def _body(cname_hbm, tsubj_hbm, ctpk_hbm, ttab_hbm, out_hbm,
          cidx_v, cidx4_v, crow_v, tidx0, tidx1, rows0, rows1, outc_v,
          csem, sem0, sem1):
    wid = lax.axis_index("s") * NC + lax.axis_index("c")
    base = wid * PER_W

    pltpu.sync_copy(cname_hbm.at[pl.ds(base, PER_W)], cidx_v)

    def shift_body(i, _):
        cidx4_v[pl.ds(i * 16, 16)] = cidx_v[pl.ds(i * 16, 16)] // 4
        return 0

    lax.fori_loop(0, PER_W // 16, shift_body, 0)
    ccopy = pltpu.async_copy(ctpk_hbm.at[cidx4_v], crow_v, csem)

    def issue(k, tidx, rows, sem):
        pltpu.sync_copy(tsubj_hbm.at[pl.ds((base + k * C) * L, C * L)], tidx)
        return pltpu.async_copy(ttab_hbm.at[tidx], rows, sem)

    def reduce_chunk(k, rows):
        iota1 = lax.iota(jnp.int32, 16)
        iota2 = 2 * iota1

        def elem_body(e, _):
            r0 = e * L
            a0, a1 = _unpack2(rows[r0, pl.ds(0, D)])
            for l in range(1, L):
                b0, b1 = _unpack2(rows[r0 + l, pl.ds(0, D)])
                a0 = a0 + b0
                a1 = a1 + b1
            ce = k * C + e
            cef = jnp.full((16,), ce, jnp.int32)
            cidv = plsc.load_gather(cidx_v, [cef])
            qv = (cidv % 4) * 32 + iota1
            o = e * (2 * D)
            c0 = plsc.load_gather(crow_v, [cef, qv])
            c1 = plsc.load_gather(crow_v, [cef, qv + 16])
            outc_v[pl.ds(o, 16)] = c0
            outc_v[pl.ds(o + 16, 16)] = c1
            plsc.store_scatter(outc_v, [o + D + iota2], a0 * INV_L)
            plsc.store_scatter(outc_v, [o + D + 1 + iota2], a1 * INV_L)
            return 0

        lax.fori_loop(0, C, elem_body, 0)
        pltpu.sync_copy(outc_v, out_hbm.at[pl.ds((base + k * C) * 2 * D,
                                                 C * 2 * D)])

    issue(0, tidx0, rows0, sem0)
    ccopy.wait()

    def pair_body(p, _):
        ka = 2 * p
        issue(ka + 1, tidx1, rows1, sem1)
        pltpu.make_async_copy(ttab_hbm.at[tidx0], rows0, sem0).wait()
        reduce_chunk(ka, rows0)

        @pl.when(p < NCHUNK // 2 - 1)
        def _():
            issue(ka + 2, tidx0, rows0, sem0)

        pltpu.make_async_copy(ttab_hbm.at[tidx1], rows1, sem1).wait()
        reduce_chunk(ka + 1, rows1)
        return 0

    lax.fori_loop(0, NCHUNK // 2, pair_body, 0)


@jax.jit
def kernel(customer_name, ticket_subject, customer_table, ticket_table):
    mesh = plsc.VectorSubcoreMesh(core_axis_name="c", subcore_axis_name="s")

    tail_flat = jnp.reshape(customer_table[NBLK * 128:], (-1,))
    tailpk = jnp.reshape(
        jnp.concatenate([tail_flat,
                         jnp.zeros((16 * 128 - 33 * 32,), jnp.float32)]),
        (16, 128))

    conv = pl.kernel(
        _conv_body,
        out_type=jax.ShapeDtypeStruct((PK_ROWS, 128), jnp.float32),
        mesh=mesh,
        scratch_types=[
            pltpu.VMEM((D, 128), jnp.float32),
            pltpu.VMEM((D, 128), jnp.float32),
            pltpu.VMEM((32, 128), jnp.float32),
            pltpu.SemaphoreType.DMA,
            pltpu.SemaphoreType.DMA,
        ],
        compiler_params=pltpu.CompilerParams(use_tc_tiling_on_sc=True,
                                             needs_layout_passes=False),
    )
    ctpk = conv(customer_table.T, tailpk)

    tt_bf = ticket_table.astype(jnp.bfloat16)
    tsubj_flat = jnp.reshape(ticket_subject, (B * L,))
    k = pl.kernel(
        _body,
        out_type=jax.ShapeDtypeStruct((B * 2 * D,), jnp.float32),
        mesh=mesh,
        scratch_types=[
            pltpu.VMEM((PER_W,), jnp.int32),
            pltpu.VMEM((PER_W,), jnp.int32),
            pltpu.VMEM((PER_W, 128), jnp.float32),
            pltpu.VMEM((C * L,), jnp.int32),
            pltpu.VMEM((C * L,), jnp.int32),
            pltpu.VMEM((C * L, D), jnp.bfloat16),
            pltpu.VMEM((C * L, D), jnp.bfloat16),
            pltpu.VMEM((C * 2 * D,), jnp.float32),
            pltpu.SemaphoreType.DMA,
            pltpu.SemaphoreType.DMA,
            pltpu.SemaphoreType.DMA,
        ],
        compiler_params=pltpu.CompilerParams(use_tc_tiling_on_sc=False,
                                             needs_layout_passes=False),
    )
    out_flat = k(customer_name, tsubj_flat, ctpk, tt_bf)
    return jnp.reshape(out_flat, (B, 2 * D))


# speedup vs baseline: 1.1679x; 1.0363x over previous
import jax
import jax.numpy as jnp
from jax import lax
from jax.experimental import pallas as pl
from jax.experimental.pallas import tpu as pltpu
from jax.experimental.pallas import tpu_sc as plsc

B = 16384
L = 50
D = 32
CUST_V = 100001
NC = 2
NS = 16
NW = NC * NS
PER_W = B // NW
C = 32
NCHUNK = PER_W // C
INV_L = 1.0 / L

NBLK = CUST_V // 128
PK_ROWS = NBLK * 32 + 16
BLK_PER_W = (NBLK + NW - 1) // NW


def _conv_body(ctT_hbm, tailpk_hbm, ctpk_hbm, cbuf0, cbuf1, obuf, dsem0,
               dsem1):
    wid = lax.axis_index("s") * NC + lax.axis_index("c")
    iota = lax.iota(jnp.int32, 16)

    def start(i, cbuf, dsem):
        blk = i * NW + wid

        @pl.when(blk < NBLK)
        def _():
            pltpu.async_copy(ctT_hbm.at[:, pl.ds(blk * 128, 128)], cbuf, dsem)

    def process(i, cbuf, dsem):
        blk = i * NW + wid

        @pl.when(blk < NBLK)
        def _():
            pltpu.make_async_copy(
                ctT_hbm.at[:, pl.ds(blk * 128, 128)], cbuf, dsem).wait()

            def grp_body(jg, _):
                jcol = jg * 16 + iota
                idx_r = jcol // 4
                idx_q0 = (jcol % 4) * 32

                def d_body(d4, _):
                    for du in range(4):
                        d = d4 * 4 + du
                        seg = cbuf[d, pl.ds(jg * 16, 16)]
                        plsc.store_scatter(obuf, [idx_r, idx_q0 + d], seg)
                    return 0

                lax.fori_loop(0, 8, d_body, 0)
                return 0

            lax.fori_loop(0, 8, grp_body, 0)
            pltpu.sync_copy(obuf, ctpk_hbm.at[pl.ds(blk * 32, 32), :])

    start(0, cbuf0, dsem0)

    def pair_body(p, _):
        ia = 2 * p
        start(ia + 1, cbuf1, dsem1)
        process(ia, cbuf0, dsem0)

        @pl.when(p < BLK_PER_W // 2)
        def _():
            start(ia + 2, cbuf0, dsem0)

        process(ia + 1, cbuf1, dsem1)
        return 0

    lax.fori_loop(0, BLK_PER_W // 2, pair_body, 0)
    process(BLK_PER_W - 1, cbuf0, dsem0)

    @pl.when(wid == 0)
    def _():
        pltpu.sync_copy(tailpk_hbm, ctpk_hbm.at[pl.ds(NBLK * 32, 16), :])


def _unpack2(x32):
    return plsc.unpack(x32, format=plsc.PackFormat.INTERLEAVED)


def _body(cname_hbm, tsubj_hbm, ctpk_hbm, ttab_hbm, out_hbm,
          cidx_v, cidx4_v, crow_v, tidx0, tidx1, rows0, rows1, outc_v,
          csem, sem0, sem1):
    wid = lax.axis_index("s") * NC + lax.axis_index("c")
    base = wid * PER_W

    pltpu.sync_copy(cname_hbm.at[pl.ds(base, PER_W)], cidx_v)

    def shift_body(i, _):
        cidx4_v[pl.ds(i * 16, 16)] = cidx_v[pl.ds(i * 16, 16)] // 4
        return 0

    lax.fori_loop(0, PER_W // 16, shift_body, 0)
    ccopy = pltpu.async_copy(ctpk_hbm.at[cidx4_v], crow_v, csem)

    def issue(k, tidx, rows, sem):
        pltpu.sync_copy(tsubj_hbm.at[pl.ds((base + k * C) * L, C * L)], tidx)
        return pltpu.async_copy(ttab_hbm.at[tidx], rows, sem)

    def reduce_chunk(k, rows):
        iota1 = lax.iota(jnp.int32, 16)
        iota2 = 2 * iota1

        def elem_body(e, _):
            r0 = e * L
            a0, a1 = _unpack2(rows[r0, pl.ds(0, D)])
            for l in range(1, L):
                b0, b1 = _unpack2(rows[r0 + l, pl.ds(0, D)])
                a0 = a0 + b0
                a1 = a1 + b1
            ce = k * C + e
            cef = jnp.full((16,), ce, jnp.int32)
            cidv = plsc.load_gather(cidx_v, [cef])
            qv = (cidv % 4) * 32 + iota1
            o = e * (2 * D)
            c0 = plsc.load_gather(crow_v, [cef, qv])
            c1 = plsc.load_gather(crow_v, [cef, qv + 16])
            outc_v[pl.ds(o, 16)] = c0
            outc_v[pl.ds(o + 16, 16)] = c1
            plsc.store_scatter(outc_v, [o + D + iota2], a0 * INV_L)
            plsc.store_scatter(outc_v, [o + D + 1 + iota2], a1 * INV_L)
            return 0

        lax.fori_loop(0, C, elem_body, 0)
        pltpu.sync_copy(outc_v, out_hbm.at[pl.ds((base + k * C) * 2 * D,
                                                 C * 2 * D)])

    issue(0, tidx0, rows0, sem0)
    ccopy.wait()

    def pair_body(p, _):
        ka = 2 * p
        issue(ka + 1, tidx1, rows1, sem1)
        pltpu.make_async_copy(ttab_hbm.at[tidx0], rows0, sem0).wait()
        reduce_chunk(ka, rows0)

        @pl.when(p < NCHUNK // 2 - 1)
        def _():
            issue(ka + 2, tidx0, rows0, sem0)

        pltpu.make_async_copy(ttab_hbm.at[tidx1], rows1, sem1).wait()
        reduce_chunk(ka + 1, rows1)
        return 0

    lax.fori_loop(0, NCHUNK // 2, pair_body, 0)


@jax.jit
def kernel(customer_name, ticket_subject, customer_table, ticket_table):
    mesh = plsc.VectorSubcoreMesh(core_axis_name="c", subcore_axis_name="s")

    tail_flat = jnp.reshape(customer_table[NBLK * 128:], (-1,))
    tailpk = jnp.reshape(
        jnp.concatenate([tail_flat,
                         jnp.zeros((16 * 128 - 33 * 32,), jnp.float32)]),
        (16, 128))

    conv = pl.kernel(
        _conv_body,
        out_type=jax.ShapeDtypeStruct((PK_ROWS, 128), jnp.float32),
        mesh=mesh,
        scratch_types=[
            pltpu.VMEM((D, 128), jnp.float32),
            pltpu.VMEM((D, 128), jnp.float32),
            pltpu.VMEM((32, 128), jnp.float32),
            pltpu.SemaphoreType.DMA,
            pltpu.SemaphoreType.DMA,
        ],
        compiler_params=pltpu.CompilerParams(use_tc_tiling_on_sc=True,
                                             needs_layout_passes=False),
    )
    ctpk = conv(customer_table.T, tailpk)

    tt_bf = ticket_table.astype(jnp.bfloat16)
    tsubj_flat = jnp.reshape(ticket_subject, (B * L,))
    k = pl.kernel(
        _body,
        out_type=jax.ShapeDtypeStruct((B * 2 * D,), jnp.float32),
        mesh=mesh,
        scratch_types=[
            pltpu.VMEM((PER_W,), jnp.int32),
            pltpu.VMEM((PER_W,), jnp.int32),
            pltpu.VMEM((PER_W, 128), jnp.float32),
            pltpu.VMEM((C * L,), jnp.int32),
            pltpu.VMEM((C * L,), jnp.int32),
            pltpu.VMEM((C * L, D), jnp.bfloat16),
            pltpu.VMEM((C * L, D), jnp.bfloat16),
            pltpu.VMEM((C * 2 * D,), jnp.float32),
            pltpu.SemaphoreType.DMA,
            pltpu.SemaphoreType.DMA,
            pltpu.SemaphoreType.DMA,
        ],
        compiler_params=pltpu.CompilerParams(use_tc_tiling_on_sc=False,
                                             needs_layout_passes=False),
    )
    out_flat = k(customer_name, tsubj_flat, ctpk, tt_bf)
    return jnp.reshape(out_flat, (B, 2 * D))


# speedup vs baseline: 1.1956x; 1.0237x over previous
import jax
import jax.numpy as jnp
from jax import lax
from jax.experimental import pallas as pl
from jax.experimental.pallas import tpu as pltpu
from jax.experimental.pallas import tpu_sc as plsc

B = 16384
L = 50
D = 32
NC = 2
NS = 16
NW = NC * NS
PER_W = B // NW
C = 32
NCHUNK = PER_W // C
INV_L = 1.0 / L


def _body(cname_hbm, tsubj_hbm, ctab_hbm, ttab_hbm, out_hbm,
          cidx_v, crow_v, tidx0, tidx1, rows0, rows1, outc_v,
          csem, sem0, sem1):
    wid = lax.axis_index("s") * NC + lax.axis_index("c")
    base = wid * PER_W

    pltpu.sync_copy(cname_hbm.at[pl.ds(base, PER_W)], cidx_v)
    ccopy = pltpu.async_copy(ctab_hbm.at[cidx_v], crow_v, csem)

    def issue(k, tidx, rows, sem):
        pltpu.sync_copy(tsubj_hbm.at[pl.ds((base + k * C) * L, C * L)], tidx)
        return pltpu.async_copy(ttab_hbm.at[tidx], rows, sem)

    def reduce_chunk(k, rows):
        def elem_body(e, _):
            r0 = e * L
            a0 = rows[r0, pl.ds(0, 16)]
            a1 = rows[r0, pl.ds(16, 16)]
            for l in range(1, L):
                a0 = a0 + rows[r0 + l, pl.ds(0, 16)]
                a1 = a1 + rows[r0 + l, pl.ds(16, 16)]
            ce = k * C + e
            outc_v[e, pl.ds(0, 16)] = crow_v[ce, pl.ds(0, 16)]
            outc_v[e, pl.ds(16, 16)] = crow_v[ce, pl.ds(16, 16)]
            outc_v[e, pl.ds(32, 16)] = a0 * INV_L
            outc_v[e, pl.ds(48, 16)] = a1 * INV_L
            return 0

        lax.fori_loop(0, C, elem_body, 0)
        pltpu.sync_copy(outc_v, out_hbm.at[pl.ds(base + k * C, C)])

    issue(0, tidx0, rows0, sem0)
    ccopy.wait()

    def pair_body(p, _):
        ka = 2 * p
        issue(ka + 1, tidx1, rows1, sem1)
        pltpu.make_async_copy(ttab_hbm.at[tidx0], rows0, sem0).wait()
        reduce_chunk(ka, rows0)

        @pl.when(p < NCHUNK // 2 - 1)
        def _():
            issue(ka + 2, tidx0, rows0, sem0)

        pltpu.make_async_copy(ttab_hbm.at[tidx1], rows1, sem1).wait()
        reduce_chunk(ka + 1, rows1)
        return 0

    lax.fori_loop(0, NCHUNK // 2, pair_body, 0)


@jax.jit
def kernel(customer_name, ticket_subject, customer_table, ticket_table):
    tsubj_flat = jnp.reshape(ticket_subject, (B * L,))
    mesh = plsc.VectorSubcoreMesh(core_axis_name="c", subcore_axis_name="s")
    k = pl.kernel(
        _body,
        out_type=jax.ShapeDtypeStruct((B, 2 * D), jnp.float32),
        mesh=mesh,
        scratch_types=[
            pltpu.VMEM((PER_W,), jnp.int32),
            pltpu.VMEM((PER_W, D), jnp.float32),
            pltpu.VMEM((C * L,), jnp.int32),
            pltpu.VMEM((C * L,), jnp.int32),
            pltpu.VMEM((C * L, D), jnp.float32),
            pltpu.VMEM((C * L, D), jnp.float32),
            pltpu.VMEM((C, 2 * D), jnp.float32),
            pltpu.SemaphoreType.DMA,
            pltpu.SemaphoreType.DMA,
            pltpu.SemaphoreType.DMA,
        ],
        compiler_params=pltpu.CompilerParams(use_tc_tiling_on_sc=False),
    )
    return k(customer_name, tsubj_flat, customer_table, ticket_table)
